# Initial kernel scaffold; baseline (speedup 1.0000x reference)
import jax
import jax.numpy as jnp
from jax import lax
from jax.experimental import pallas as pl
from jax.experimental.pallas import tpu as pltpu
from jax.experimental.pallas import tpu_sc as plsc

N = 10000
D = 128
E = 320000
NPAD = 10240
NC = 2
NT = 16
RPT = NPAD // NT
CH = 128
NCHUNKS = E // CH
FCH = 80

f32 = jnp.float32
i32 = jnp.int32


def _zero_2d(ref, nrows, ncols):
    z = jnp.zeros((16,), f32)

    def body(i, _):
        for g in range(ncols // 16):
            ref[i, pl.ds(16 * g, 16)] = z
        return 0

    lax.fori_loop(0, nrows, body, 0)


def _fill_2d(ref, nrows, ncols, val):
    v = jnp.full((16,), val, f32)

    def body(i, _):
        for g in range(ncols // 16):
            ref[i, pl.ds(16 * g, 16)] = v
        return 0

    lax.fori_loop(0, nrows, body, 0)


def _n_edge_trips(s):
    return (NCHUNKS - s + NT - 1) // NT


def _agg_pass(s, e_hbm, table_hbm, acc, srcb, dstb, rows, sem, degt=None,
              ones16=None):
    def body(i, _):
        c = s + NT * i
        off = c * CH
        pltpu.sync_copy(e_hbm.at[0, pl.ds(off, CH)], srcb)
        pltpu.sync_copy(e_hbm.at[1, pl.ds(off, CH)], dstb)
        if degt is not None:
            pltpu.sync_copy(ones16, degt.at[dstb], add=True)
        pltpu.async_copy(table_hbm.at[srcb], rows, sem).wait()
        pltpu.sync_copy(rows, acc.at[dstb], add=True)
        return 0

    lax.fori_loop(0, _n_edge_trips(s), body, 0)


def _finalize(s, acc, base_hbm, out_hbm, col0, invb, abuf, xbuf):
    nch = jnp.where(s == NT - 1, 5, 8)

    def chunk(k, _):
        r0 = s * RPT + FCH * k
        pltpu.sync_copy(acc.at[pl.ds(r0, FCH)], abuf)
        pltpu.sync_copy(base_hbm.at[pl.ds(r0, FCH)], xbuf)

        def row(r, _):
            iv = plsc.load_gather(invb, [jnp.full((16,), FCH * k + r, i32)])
            for g in range(D // 16):
                sl = pl.ds(16 * g, 16)
                a = abuf[r, sl]
                xv = xbuf[r, sl]
                y = xv + a * iv
                abuf[r, sl] = jnp.where(y >= 0.0, y, 0.01 * y)
            return 0

        lax.fori_loop(0, FCH, row, 0)
        if col0 is None:
            pltpu.sync_copy(abuf, out_hbm.at[pl.ds(r0, FCH)])
        else:
            pltpu.sync_copy(abuf, out_hbm.at[pl.ds(r0, FCH), pl.ds(col0, D)])
        return 0

    lax.fori_loop(0, nch, chunk, 0)


def _layer1_core(s, x_hbm, e_hbm, h_hbm, inv_hbm, acc, degt, srcb, dstb, rows,
                 abuf, xbuf, invb, ones16, dbuf, sem):
    _zero_2d(rows, CH, D)
    for k in range(RPT // CH):
        pltpu.sync_copy(rows, acc.at[pl.ds(s * RPT + CH * k, CH)])
    _zero_2d(dbuf, CH, 16)
    for k in range(RPT // CH):
        pltpu.sync_copy(dbuf, degt.at[pl.ds(s * RPT + CH * k, CH)])
    _fill_2d(ones16, CH, 16, 1.0)
    plsc.subcore_barrier()

    _agg_pass(s, e_hbm, x_hbm, acc, srcb, dstb, rows, sem, degt=degt,
              ones16=ones16)
    plsc.subcore_barrier()

    for k in range(RPT // CH):
        pltpu.sync_copy(degt.at[pl.ds(s * RPT + CH * k, CH)], dbuf)

        def blk(b, _):
            ridx = lax.iota(i32, 16) + 16 * b
            d16 = plsc.load_gather(dbuf, [ridx, jnp.zeros((16,), i32)])
            inv16 = 1.0 / jnp.maximum(d16, 1.0)
            invb[pl.ds(CH * k + 16 * b, 16)] = inv16
            return 0

        lax.fori_loop(0, CH // 16, blk, 0)
    pltpu.sync_copy(invb, inv_hbm.at[pl.ds(s * RPT, RPT)])

    _finalize(s, acc, x_hbm, h_hbm, None, invb, abuf, xbuf)


def _layer1_body(x_hbm, e1_hbm, e2_hbm, h1_hbm, h2_hbm, inv1_hbm, inv2_hbm,
                 srcb, dstb, rows, abuf, xbuf, invb, ones16, dbuf, acc, degt,
                 sem):
    c = lax.axis_index("c")
    s = lax.axis_index("s")

    @pl.when(c == 0)
    def _():
        _layer1_core(s, x_hbm, e1_hbm, h1_hbm, inv1_hbm, acc, degt, srcb,
                     dstb, rows, abuf, xbuf, invb, ones16, dbuf, sem)

    @pl.when(c == 1)
    def _():
        _layer1_core(s, x_hbm, e2_hbm, h2_hbm, inv2_hbm, acc, degt, srcb,
                     dstb, rows, abuf, xbuf, invb, ones16, dbuf, sem)


def _layer2_core(s, e_hbm, inv_hbm, h1_hbm, h2_hbm, out_hbm, cbase, acc, srcb,
                 dstb, rows, abuf, xbuf, invb, sem):
    pltpu.sync_copy(inv_hbm.at[pl.ds(s * RPT, RPT)], invb)
    for p, table in enumerate((h1_hbm, h2_hbm)):
        _zero_2d(rows, CH, D)
        for k in range(RPT // CH):
            pltpu.sync_copy(rows, acc.at[pl.ds(s * RPT + CH * k, CH)])
        plsc.subcore_barrier()
        _agg_pass(s, e_hbm, table, acc, srcb, dstb, rows, sem)
        plsc.subcore_barrier()
        _finalize(s, acc, table, out_hbm, cbase + D * p, invb, abuf, xbuf)
        plsc.subcore_barrier()


def _layer2_body(h1_hbm, h2_hbm, e1_hbm, e2_hbm, inv1_hbm, inv2_hbm, out_hbm,
                 srcb, dstb, rows, abuf, xbuf, invb, acc, sem):
    c = lax.axis_index("c")
    s = lax.axis_index("s")

    @pl.when(c == 0)
    def _():
        _layer2_core(s, e1_hbm, inv1_hbm, h1_hbm, h2_hbm, out_hbm, 0, acc,
                     srcb, dstb, rows, abuf, xbuf, invb, sem)

    @pl.when(c == 1)
    def _():
        _layer2_core(s, e2_hbm, inv2_hbm, h1_hbm, h2_hbm, out_hbm, 2 * D, acc,
                     srcb, dstb, rows, abuf, xbuf, invb, sem)


def _mesh():
    return plsc.VectorSubcoreMesh(core_axis_name="c", subcore_axis_name="s",
                                  num_cores=NC, num_subcores=NT)


@jax.jit
def kernel(node_features, edge_index1, edge_index2):
    layer1 = pl.kernel(
        _layer1_body,
        out_type=[
            jax.ShapeDtypeStruct((N, D), f32),
            jax.ShapeDtypeStruct((N, D), f32),
            jax.ShapeDtypeStruct((NPAD,), f32),
            jax.ShapeDtypeStruct((NPAD,), f32),
        ],
        mesh=_mesh(),
        scratch_types=[
            pltpu.VMEM((CH,), i32),
            pltpu.VMEM((CH,), i32),
            pltpu.VMEM((CH, D), f32),
            pltpu.VMEM((FCH, D), f32),
            pltpu.VMEM((FCH, D), f32),
            pltpu.VMEM((RPT,), f32),
            pltpu.VMEM((CH, 16), f32),
            pltpu.VMEM((CH, 16), f32),
            pltpu.VMEM_SHARED((NPAD, D), f32),
            pltpu.VMEM_SHARED((NPAD, 16), f32),
            pltpu.SemaphoreType.DMA,
        ],
        name="gin_layer1",
    )
    h1, h2, inv1, inv2 = layer1(node_features, edge_index1, edge_index2)

    layer2 = pl.kernel(
        _layer2_body,
        out_type=jax.ShapeDtypeStruct((N, 4 * D), f32),
        mesh=_mesh(),
        scratch_types=[
            pltpu.VMEM((CH,), i32),
            pltpu.VMEM((CH,), i32),
            pltpu.VMEM((CH, D), f32),
            pltpu.VMEM((FCH, D), f32),
            pltpu.VMEM((FCH, D), f32),
            pltpu.VMEM((RPT,), f32),
            pltpu.VMEM_SHARED((NPAD, D), f32),
            pltpu.SemaphoreType.DMA,
        ],
        name="gin_layer2",
    )
    return layer2(h1, h2, edge_index1, edge_index2, inv1, inv2)


# speedup vs baseline: 4.2033x; 4.2033x over previous
import jax
import jax.numpy as jnp
from jax import lax
from jax.experimental import pallas as pl
from jax.experimental.pallas import tpu as pltpu
from jax.experimental.pallas import tpu_sc as plsc

N = 10000
D = 128
E = 320000
NPAD = 10240
NC = 2
NT = 16
RPT = NPAD // NT
CH = 128
NCHUNKS = E // CH
FCH = 16
NRV = NPAD // 128

f32 = jnp.float32
i32 = jnp.int32


def _n_edge_trips(s):
    return (NCHUNKS - s + NT - 1) // NT


def _agg_pass(s, src_hbm, dst_hbm, table_hbm, acc, srcb, dstb, rows, sem,
              degv=None):
    ones = jnp.full((16,), 1.0, f32)

    def body(i, _):
        off = (s + NT * i) * CH
        pltpu.sync_copy(src_hbm.at[pl.ds(off, CH)], srcb)
        pltpu.sync_copy(dst_hbm.at[pl.ds(off, CH)], dstb)
        pltpu.async_copy(table_hbm.at[srcb], rows, sem).wait()
        if degv is not None:
            for t in range(CH // 16):
                dv = dstb[pl.ds(16 * t, 16)]
                plsc.addupdate_scatter(degv, [dv], ones)
        pltpu.sync_copy(rows, acc.at[dstb], add=True)
        return 0

    lax.fori_loop(0, _n_edge_trips(s), body, 0)


def _finalize(s, acc, base_hbm, out_hbm, col0, rows, ibuf, fidx, dfl=None,
              inv_hbm=None):
    nch = jnp.where(s == NT - 1, 25, 40)
    zero16 = jnp.zeros((16,), i32)

    def chunk(k, _):
        r0 = s * RPT + FCH * k
        fidx[pl.ds(0, 16)] = lax.iota(i32, 16) + r0
        pltpu.sync_copy(acc.at[fidx], rows.at[pl.ds(0, FCH)])
        pltpu.sync_copy(base_hbm.at[pl.ds(r0, FCH)], rows.at[pl.ds(FCH, FCH)])
        if dfl is None:
            pltpu.sync_copy(inv_hbm.at[pl.ds(r0, FCH)], ibuf)

        def row(r, _):
            if dfl is not None:
                dv = plsc.load_gather(dfl, [zero16 + (FCH * k + r)])
                iv = 1.0 / jnp.maximum(dv, 1.0)
                ibuf[r, pl.ds(0, 16)] = iv
            else:
                iv = ibuf[r, pl.ds(0, 16)]
            for g in range(D // 16):
                sl = pl.ds(16 * g, 16)
                y = rows[FCH + r, sl] + rows[r, sl] * iv
                rows[r, sl] = jnp.where(y >= 0.0, y, 0.01 * y)
            return 0

        lax.fori_loop(0, FCH, row, 0)
        if col0 is None:
            pltpu.sync_copy(rows.at[pl.ds(0, FCH)], out_hbm.at[pl.ds(r0, FCH)])
        else:
            pltpu.sync_copy(rows.at[pl.ds(0, FCH)],
                            out_hbm.at[pl.ds(r0, FCH), pl.ds(col0, D)])
        if dfl is not None:
            pltpu.sync_copy(ibuf, inv_hbm.at[pl.ds(r0, FCH)])
        return 0

    lax.fori_loop(0, nch, chunk, 0)


def _fill_zero_rows(rows):
    z = jnp.zeros((16,), f32)

    def body(i, _):
        for g in range(D // 16):
            rows[i, pl.ds(16 * g, 16)] = z
        return 0

    lax.fori_loop(0, CH, body, 0)


def _zero_acc(s, acc, rows, srcb):
    _fill_zero_rows(rows)
    for k in range(RPT // CH):
        r0 = s * RPT + CH * k

        def put_idx(t, _):
            srcb[pl.ds(16 * t, 16)] = lax.iota(i32, 16) + (r0 + 16 * t)
            return 0

        lax.fori_loop(0, CH // 16, put_idx, 0)
        pltpu.sync_copy(rows, acc.at[srcb])


def _layer1_body(x_hbm, src1_hbm, dst1_hbm, src2_hbm, dst2_hbm,
                 h1_hbm, h2_hbm, inv1_hbm, inv2_hbm,
                 srcb, dstb, rows, ibuf, fidx, ridx, gidx, degv, dfl, dgb,
                 acc, degsh, sem):
    c = lax.axis_index("c")
    s = lax.axis_index("s")
    z = jnp.zeros((16,), f32)

    _zero_acc(s, acc, rows, srcb)

    def put_ridx(t, _):
        ridx[pl.ds(16 * t, 16)] = lax.iota(i32, 16) + 16 * t
        return 0

    lax.fori_loop(0, 128 // 16, put_ridx, 0)
    pltpu.sync_copy(rows, degsh.at[ridx])

    def zdeg(i, _):
        degv[pl.ds(16 * i, 16)] = z
        return 0

    lax.fori_loop(0, NPAD // 16, zdeg, 0)
    plsc.subcore_barrier()

    @pl.when(c == 0)
    def _():
        _agg_pass(s, src1_hbm, dst1_hbm, x_hbm, acc, srcb, dstb, rows, sem,
                  degv=degv)

    @pl.when(c == 1)
    def _():
        _agg_pass(s, src2_hbm, dst2_hbm, x_hbm, acc, srcb, dstb, rows, sem,
                  degv=degv)

    plsc.subcore_barrier()

    def cpdeg(j, _):
        for g in range(D // 16):
            rows[j, pl.ds(16 * g, 16)] = degv[pl.ds(128 * j + 16 * g, 16)]
        return 0

    lax.fori_loop(0, NRV, cpdeg, 0)
    pltpu.sync_copy(rows, degsh.at[ridx], add=True)
    plsc.subcore_barrier()

    gidx[pl.ds(0, 16)] = jnp.minimum(lax.iota(i32, 16), NRV // NT - 1) + \
        (NRV // NT) * s
    pltpu.sync_copy(degsh.at[gidx], dgb)

    def flat(j, _):
        for g in range(D // 16):
            dfl[pl.ds(128 * j + 16 * g, 16)] = dgb[j, pl.ds(16 * g, 16)]
        return 0

    lax.fori_loop(0, NRV // NT, flat, 0)

    @pl.when(c == 0)
    def _():
        _finalize(s, acc, x_hbm, h1_hbm, None, rows, ibuf, fidx, dfl=dfl,
                  inv_hbm=inv1_hbm)

    @pl.when(c == 1)
    def _():
        _finalize(s, acc, x_hbm, h2_hbm, None, rows, ibuf, fidx, dfl=dfl,
                  inv_hbm=inv2_hbm)


def _layer2_body(h1_hbm, h2_hbm, src1_hbm, dst1_hbm, src2_hbm, dst2_hbm,
                 inv1_hbm, inv2_hbm, out_hbm,
                 srcb, dstb, rows, ibuf, fidx, acc, sem):
    c = lax.axis_index("c")
    s = lax.axis_index("s")

    for p, table in enumerate((h1_hbm, h2_hbm)):
        _zero_acc(s, acc, rows, srcb)
        plsc.subcore_barrier()

        @pl.when(c == 0)
        def _():
            _agg_pass(s, src1_hbm, dst1_hbm, table, acc, srcb, dstb, rows,
                      sem)

        @pl.when(c == 1)
        def _():
            _agg_pass(s, src2_hbm, dst2_hbm, table, acc, srcb, dstb, rows,
                      sem)

        plsc.subcore_barrier()

        @pl.when(c == 0)
        def _():
            _finalize(s, acc, table, out_hbm, D * p, rows, ibuf, fidx,
                      inv_hbm=inv1_hbm)

        @pl.when(c == 1)
        def _():
            _finalize(s, acc, table, out_hbm, 2 * D + D * p, rows, ibuf, fidx,
                      inv_hbm=inv2_hbm)

        plsc.subcore_barrier()


def _mesh():
    return plsc.VectorSubcoreMesh(core_axis_name="c", subcore_axis_name="s",
                                  num_cores=NC, num_subcores=NT)


_PARAMS = pltpu.CompilerParams(needs_layout_passes=False)


@jax.jit
def kernel(node_features, edge_index1, edge_index2):
    src1, dst1 = edge_index1[0], edge_index1[1]
    src2, dst2 = edge_index2[0], edge_index2[1]

    layer1 = pl.kernel(
        _layer1_body,
        out_type=[
            jax.ShapeDtypeStruct((N, D), f32),
            jax.ShapeDtypeStruct((N, D), f32),
            jax.ShapeDtypeStruct((NPAD, 16), f32),
            jax.ShapeDtypeStruct((NPAD, 16), f32),
        ],
        mesh=_mesh(),
        compiler_params=_PARAMS,
        scratch_types=[
            pltpu.VMEM((CH,), i32),
            pltpu.VMEM((CH,), i32),
            pltpu.VMEM((CH, D), f32),
            pltpu.VMEM((FCH, 16), f32),
            pltpu.VMEM((16,), i32),
            pltpu.VMEM((128,), i32),
            pltpu.VMEM((16,), i32),
            pltpu.VMEM((NPAD,), f32),
            pltpu.VMEM((RPT,), f32),
            pltpu.VMEM((16, D), f32),
            pltpu.VMEM_SHARED((NPAD, D), f32),
            pltpu.VMEM_SHARED((128, D), f32),
            pltpu.SemaphoreType.DMA,
        ],
        name="gin_layer1",
    )
    h1, h2, inv1, inv2 = layer1(node_features, src1, dst1, src2, dst2)

    layer2 = pl.kernel(
        _layer2_body,
        out_type=jax.ShapeDtypeStruct((N, 4 * D), f32),
        mesh=_mesh(),
        compiler_params=_PARAMS,
        scratch_types=[
            pltpu.VMEM((CH,), i32),
            pltpu.VMEM((CH,), i32),
            pltpu.VMEM((CH, D), f32),
            pltpu.VMEM((FCH, 16), f32),
            pltpu.VMEM((16,), i32),
            pltpu.VMEM_SHARED((NPAD, D), f32),
            pltpu.SemaphoreType.DMA,
        ],
        name="gin_layer2",
    )
    return layer2(h1, h2, src1, dst1, src2, dst2, inv1, inv2)


# speedup vs baseline: 4.7896x; 1.1395x over previous
import jax
import jax.numpy as jnp
from jax import lax
from jax.experimental import pallas as pl
from jax.experimental.pallas import tpu as pltpu
from jax.experimental.pallas import tpu_sc as plsc

N = 10000
D = 128
E = 320000
NPAD = 10240
NC = 2
NT = 16
RPT = NPAD // NT
CH = 64
NCHUNKS = E // CH
FCH = 16
NRV = NPAD // 128

f32 = jnp.float32
i32 = jnp.int32


def _n_edge_trips(s):
    return (NCHUNKS - s + NT - 1) // NT


def _agg_pass(s, src_hbm, dst_hbm, table_hbm, acc,
              srcb0, dstb0, rows0, sem0, srcb1, dstb1, rows1, sem1,
              degv=None):
    ones = jnp.full((16,), 1.0, f32)
    trips = _n_edge_trips(s)
    bufs = ((srcb0, dstb0, rows0, sem0), (srcb1, dstb1, rows1, sem1))

    def start(c, sb, db, rb, sm):
        off = (s + NT * c) * CH
        pltpu.sync_copy(src_hbm.at[pl.ds(off, CH)], sb)
        pltpu.sync_copy(dst_hbm.at[pl.ds(off, CH)], db)
        pltpu.async_copy(table_hbm.at[sb], rb, sm)

    def drain(sb, db, rb, sm):
        pltpu.make_async_copy(table_hbm.at[sb], rb, sm).wait()
        if degv is not None:
            for t in range(CH // 16):
                dv = db[pl.ds(16 * t, 16)]
                plsc.addupdate_scatter(degv, [dv], ones)
        pltpu.sync_copy(rb, acc.at[db], add=True)

    start(0, *bufs[0])

    def body(i2, _):
        a = 2 * i2

        @pl.when(a + 1 < trips)
        def _():
            start(a + 1, *bufs[1])

        drain(*bufs[0])

        @pl.when(a + 2 < trips)
        def _():
            start(a + 2, *bufs[0])

        @pl.when(a + 1 < trips)
        def _():
            drain(*bufs[1])

        return 0

    lax.fori_loop(0, (trips + 1) // 2, body, 0)


def _finalize(s, acc, base_hbm, out_hbm, col0, rows, ibuf, fidx, dfl=None,
              inv_hbm=None):
    nch = jnp.where(s == NT - 1, 25, 40)
    zero16 = jnp.zeros((16,), i32)

    def chunk(k, _):
        r0 = s * RPT + FCH * k
        fidx[pl.ds(0, 16)] = lax.iota(i32, 16) + r0
        pltpu.sync_copy(acc.at[fidx], rows.at[pl.ds(0, FCH)])
        pltpu.sync_copy(base_hbm.at[pl.ds(r0, FCH)], rows.at[pl.ds(FCH, FCH)])
        if dfl is None:
            pltpu.sync_copy(inv_hbm.at[pl.ds(r0, FCH)], ibuf)

        def row(r, _):
            if dfl is not None:
                dv = plsc.load_gather(dfl, [zero16 + (FCH * k + r)])
                iv = 1.0 / jnp.maximum(dv, 1.0)
                ibuf[r, pl.ds(0, 16)] = iv
            else:
                iv = ibuf[r, pl.ds(0, 16)]
            for g in range(D // 16):
                sl = pl.ds(16 * g, 16)
                y = rows[FCH + r, sl] + rows[r, sl] * iv
                rows[r, sl] = jnp.where(y >= 0.0, y, 0.01 * y)
            return 0

        lax.fori_loop(0, FCH, row, 0)
        if col0 is None:
            pltpu.sync_copy(rows.at[pl.ds(0, FCH)], out_hbm.at[pl.ds(r0, FCH)])
        else:
            pltpu.sync_copy(rows.at[pl.ds(0, FCH)],
                            out_hbm.at[pl.ds(r0, FCH), pl.ds(col0, D)])
        if dfl is not None:
            pltpu.sync_copy(ibuf, inv_hbm.at[pl.ds(r0, FCH)])
        return 0

    lax.fori_loop(0, nch, chunk, 0)


def _fill_zero_rows(rows, nrows):
    z = jnp.zeros((16,), f32)

    def body(i, _):
        for g in range(D // 16):
            rows[i, pl.ds(16 * g, 16)] = z
        return 0

    lax.fori_loop(0, nrows, body, 0)


def _zero_acc(s, acc, rows, srcb):
    _fill_zero_rows(rows, CH)
    for k in range(RPT // CH):
        r0 = s * RPT + CH * k

        def put_idx(t, _):
            srcb[pl.ds(16 * t, 16)] = lax.iota(i32, 16) + (r0 + 16 * t)
            return 0

        lax.fori_loop(0, CH // 16, put_idx, 0)
        pltpu.sync_copy(rows, acc.at[srcb])


def _layer1_body(x_hbm, src1_hbm, dst1_hbm, src2_hbm, dst2_hbm,
                 h1_hbm, h2_hbm, inv1_hbm, inv2_hbm,
                 srcb0, dstb0, rows0, srcb1, dstb1, rows1,
                 ibuf, fidx, ridx, gidx, degv, dfl, dgb,
                 acc, degsh, sem0, sem1):
    c = lax.axis_index("c")
    s = lax.axis_index("s")
    z = jnp.zeros((16,), f32)

    _zero_acc(s, acc, rows0, srcb0)
    _fill_zero_rows(rows1, CH)

    def put_ridx(t, _):
        ridx[pl.ds(16 * t, 16)] = lax.iota(i32, 16) + 16 * t
        return 0

    lax.fori_loop(0, CH // 16, put_ridx, 0)
    pltpu.sync_copy(rows0, degsh.at[ridx])

    def put_ridx2(t, _):
        ridx[pl.ds(16 * t, 16)] = lax.iota(i32, 16) + (CH + 16 * t)
        return 0

    lax.fori_loop(0, CH // 16, put_ridx2, 0)
    pltpu.sync_copy(rows0, degsh.at[ridx])

    def zdeg(i, _):
        degv[pl.ds(16 * i, 16)] = z
        return 0

    lax.fori_loop(0, NPAD // 16, zdeg, 0)
    plsc.subcore_barrier()

    @pl.when(c == 0)
    def _():
        _agg_pass(s, src1_hbm, dst1_hbm, x_hbm, acc,
                  srcb0, dstb0, rows0, sem0, srcb1, dstb1, rows1, sem1,
                  degv=degv)

    @pl.when(c == 1)
    def _():
        _agg_pass(s, src2_hbm, dst2_hbm, x_hbm, acc,
                  srcb0, dstb0, rows0, sem0, srcb1, dstb1, rows1, sem1,
                  degv=degv)

    plsc.subcore_barrier()

    def cpdeg(j, _):
        for g in range(D // 16):
            rows0[j, pl.ds(16 * g, 16)] = degv[pl.ds(128 * j + 16 * g, 16)]
        return 0

    lax.fori_loop(0, CH, cpdeg, 0)
    lax.fori_loop(0, CH // 16, put_ridx, 0)
    pltpu.sync_copy(rows0, degsh.at[ridx], add=True)

    def cpdeg2(j, _):
        for g in range(D // 16):
            rows0[j, pl.ds(16 * g, 16)] = degv[pl.ds(128 * (CH + j) + 16 * g,
                                                     16)]
        return 0

    lax.fori_loop(0, NRV - CH, cpdeg2, 0)

    def zrows(j, _):
        for g in range(D // 16):
            rows0[(NRV - CH) + j, pl.ds(16 * g, 16)] = z
        return 0

    lax.fori_loop(0, CH - (NRV - CH), zrows, 0)
    lax.fori_loop(0, CH // 16, put_ridx2, 0)
    pltpu.sync_copy(rows0, degsh.at[ridx], add=True)
    plsc.subcore_barrier()

    gidx[pl.ds(0, 16)] = jnp.minimum(lax.iota(i32, 16), NRV // NT - 1) + \
        (NRV // NT) * s
    pltpu.sync_copy(degsh.at[gidx], dgb)

    def flat(j, _):
        for g in range(D // 16):
            dfl[pl.ds(128 * j + 16 * g, 16)] = dgb[j, pl.ds(16 * g, 16)]
        return 0

    lax.fori_loop(0, NRV // NT, flat, 0)

    @pl.when(c == 0)
    def _():
        _finalize(s, acc, x_hbm, h1_hbm, None, rows0, ibuf, fidx, dfl=dfl,
                  inv_hbm=inv1_hbm)

    @pl.when(c == 1)
    def _():
        _finalize(s, acc, x_hbm, h2_hbm, None, rows0, ibuf, fidx, dfl=dfl,
                  inv_hbm=inv2_hbm)


def _layer2_body(h1_hbm, h2_hbm, src1_hbm, dst1_hbm, src2_hbm, dst2_hbm,
                 inv1_hbm, inv2_hbm, out_hbm,
                 srcb0, dstb0, rows0, srcb1, dstb1, rows1, ibuf, fidx,
                 acc, sem0, sem1):
    c = lax.axis_index("c")
    s = lax.axis_index("s")

    for p, table in enumerate((h1_hbm, h2_hbm)):
        _zero_acc(s, acc, rows0, srcb0)
        plsc.subcore_barrier()

        @pl.when(c == 0)
        def _():
            _agg_pass(s, src1_hbm, dst1_hbm, table, acc,
                      srcb0, dstb0, rows0, sem0, srcb1, dstb1, rows1, sem1)

        @pl.when(c == 1)
        def _():
            _agg_pass(s, src2_hbm, dst2_hbm, table, acc,
                      srcb0, dstb0, rows0, sem0, srcb1, dstb1, rows1, sem1)

        plsc.subcore_barrier()

        @pl.when(c == 0)
        def _():
            _finalize(s, acc, table, out_hbm, D * p, rows0, ibuf, fidx,
                      inv_hbm=inv1_hbm)

        @pl.when(c == 1)
        def _():
            _finalize(s, acc, table, out_hbm, 2 * D + D * p, rows0, ibuf,
                      fidx, inv_hbm=inv2_hbm)

        plsc.subcore_barrier()


def _mesh():
    return plsc.VectorSubcoreMesh(core_axis_name="c", subcore_axis_name="s",
                                  num_cores=NC, num_subcores=NT)


_PARAMS = pltpu.CompilerParams(needs_layout_passes=False)


@jax.jit
def kernel(node_features, edge_index1, edge_index2):
    src1, dst1 = edge_index1[0], edge_index1[1]
    src2, dst2 = edge_index2[0], edge_index2[1]

    layer1 = pl.kernel(
        _layer1_body,
        out_type=[
            jax.ShapeDtypeStruct((N, D), f32),
            jax.ShapeDtypeStruct((N, D), f32),
            jax.ShapeDtypeStruct((NPAD, 16), f32),
            jax.ShapeDtypeStruct((NPAD, 16), f32),
        ],
        mesh=_mesh(),
        compiler_params=_PARAMS,
        scratch_types=[
            pltpu.VMEM((CH,), i32),
            pltpu.VMEM((CH,), i32),
            pltpu.VMEM((CH, D), f32),
            pltpu.VMEM((CH,), i32),
            pltpu.VMEM((CH,), i32),
            pltpu.VMEM((CH, D), f32),
            pltpu.VMEM((FCH, 16), f32),
            pltpu.VMEM((16,), i32),
            pltpu.VMEM((CH,), i32),
            pltpu.VMEM((16,), i32),
            pltpu.VMEM((NPAD,), f32),
            pltpu.VMEM((RPT,), f32),
            pltpu.VMEM((16, D), f32),
            pltpu.VMEM_SHARED((NPAD, D), f32),
            pltpu.VMEM_SHARED((128, D), f32),
            pltpu.SemaphoreType.DMA,
            pltpu.SemaphoreType.DMA,
        ],
        name="gin_layer1",
    )
    h1, h2, inv1, inv2 = layer1(node_features, src1, dst1, src2, dst2)

    layer2 = pl.kernel(
        _layer2_body,
        out_type=jax.ShapeDtypeStruct((N, 4 * D), f32),
        mesh=_mesh(),
        compiler_params=_PARAMS,
        scratch_types=[
            pltpu.VMEM((CH,), i32),
            pltpu.VMEM((CH,), i32),
            pltpu.VMEM((CH, D), f32),
            pltpu.VMEM((CH,), i32),
            pltpu.VMEM((CH,), i32),
            pltpu.VMEM((CH, D), f32),
            pltpu.VMEM((FCH, 16), f32),
            pltpu.VMEM((16,), i32),
            pltpu.VMEM_SHARED((NPAD, D), f32),
            pltpu.SemaphoreType.DMA,
            pltpu.SemaphoreType.DMA,
        ],
        name="gin_layer2",
    )
    return layer2(h1, h2, src1, dst1, src2, dst2, inv1, inv2)


# speedup vs baseline: 5.8136x; 1.2138x over previous
import jax
import jax.numpy as jnp
from jax import lax
from jax.experimental import pallas as pl
from jax.experimental.pallas import tpu as pltpu
from jax.experimental.pallas import tpu_sc as plsc

N = 10000
D = 128
E = 320000
NPAD = 10240
NC = 2
NT = 16
RPT = NPAD // NT
CH = 64
CH2 = 128
FCH = 16
NRV = NPAD // 128

f32 = jnp.float32
i32 = jnp.int32


def _n_edge_trips(s, ch):
    return (E // ch - s + NT - 1) // NT


def _agg_pass(s, src_hbm, dst_hbm, table_hbm, acc,
              srcb0, dstb0, rows0, sem0, srcb1, dstb1, rows1, sem1,
              ch=CH, degv=None):
    ones = jnp.full((16,), 1.0, f32)
    trips = _n_edge_trips(s, ch)
    bufs = ((srcb0, dstb0, rows0, sem0), (srcb1, dstb1, rows1, sem1))

    def start(c, sb, db, rb, sm):
        off = (s + NT * c) * ch
        pltpu.sync_copy(src_hbm.at[pl.ds(off, ch)], sb)
        pltpu.sync_copy(dst_hbm.at[pl.ds(off, ch)], db)
        pltpu.async_copy(table_hbm.at[sb], rb, sm)

    def drain(sb, db, rb, sm):
        pltpu.make_async_copy(table_hbm.at[sb], rb, sm).wait()
        if degv is not None:
            for t in range(ch // 16):
                dv = db[pl.ds(16 * t, 16)]
                plsc.addupdate_scatter(degv, [dv], ones)
        pltpu.sync_copy(rb, acc.at[db], add=True)

    start(0, *bufs[0])

    def body(i2, _):
        a = 2 * i2

        @pl.when(a + 1 < trips)
        def _():
            start(a + 1, *bufs[1])

        drain(*bufs[0])

        @pl.when(a + 2 < trips)
        def _():
            start(a + 2, *bufs[0])

        @pl.when(a + 1 < trips)
        def _():
            drain(*bufs[1])

        return 0

    lax.fori_loop(0, (trips + 1) // 2, body, 0)


def _finalize(s, acc, base_hbm, out_hbm, col0, rows, ibuf, fidx, dfl=None,
              inv_hbm=None):
    nch = jnp.where(s == NT - 1, 25, 40)
    zero16 = jnp.zeros((16,), i32)

    def chunk(k, _):
        r0 = s * RPT + FCH * k
        fidx[pl.ds(0, 16)] = lax.iota(i32, 16) + r0
        pltpu.sync_copy(acc.at[fidx], rows.at[pl.ds(0, FCH)])
        pltpu.sync_copy(base_hbm.at[pl.ds(r0, FCH)], rows.at[pl.ds(FCH, FCH)])
        if dfl is None:
            pltpu.sync_copy(inv_hbm.at[pl.ds(r0, FCH)], ibuf)

        def row(r, _):
            if dfl is not None:
                dv = plsc.load_gather(dfl, [zero16 + (FCH * k + r)])
                iv = 1.0 / jnp.maximum(dv, 1.0)
                ibuf[r, pl.ds(0, 16)] = iv
            else:
                iv = ibuf[r, pl.ds(0, 16)]
            for g in range(D // 16):
                sl = pl.ds(16 * g, 16)
                y = rows[FCH + r, sl] + rows[r, sl] * iv
                rows[r, sl] = jnp.where(y >= 0.0, y, 0.01 * y)
            return 0

        lax.fori_loop(0, FCH, row, 0)
        if col0 is None:
            pltpu.sync_copy(rows.at[pl.ds(0, FCH)], out_hbm.at[pl.ds(r0, FCH)])
        else:
            pltpu.sync_copy(rows.at[pl.ds(0, FCH)],
                            out_hbm.at[pl.ds(r0, FCH), pl.ds(col0, D)])
        if dfl is not None:
            pltpu.sync_copy(ibuf, inv_hbm.at[pl.ds(r0, FCH)])
        return 0

    lax.fori_loop(0, nch, chunk, 0)


def _fill_zero_rows(rows, nrows):
    z = jnp.zeros((16,), f32)

    def body(i, _):
        for g in range(D // 16):
            rows[i, pl.ds(16 * g, 16)] = z
        return 0

    lax.fori_loop(0, nrows, body, 0)


def _zero_acc(s, acc, rows, srcb, ch=CH):
    _fill_zero_rows(rows, ch)
    for k in range(RPT // ch):
        r0 = s * RPT + ch * k

        def put_idx(t, _):
            srcb[pl.ds(16 * t, 16)] = lax.iota(i32, 16) + (r0 + 16 * t)
            return 0

        lax.fori_loop(0, ch // 16, put_idx, 0)
        pltpu.sync_copy(rows, acc.at[srcb])


def _layer1_body(x_hbm, src1_hbm, dst1_hbm, src2_hbm, dst2_hbm,
                 h1_hbm, h2_hbm, inv1_hbm, inv2_hbm,
                 srcb0, dstb0, rows0, srcb1, dstb1, rows1,
                 ibuf, fidx, ridx, gidx, degv, dfl, dgb,
                 acc, degsh, sem0, sem1):
    c = lax.axis_index("c")
    s = lax.axis_index("s")
    z = jnp.zeros((16,), f32)

    _zero_acc(s, acc, rows0, srcb0)
    _fill_zero_rows(rows1, CH)

    def put_ridx(t, _):
        ridx[pl.ds(16 * t, 16)] = lax.iota(i32, 16) + 16 * t
        return 0

    lax.fori_loop(0, CH // 16, put_ridx, 0)
    pltpu.sync_copy(rows0, degsh.at[ridx])

    def put_ridx2(t, _):
        ridx[pl.ds(16 * t, 16)] = lax.iota(i32, 16) + (CH + 16 * t)
        return 0

    lax.fori_loop(0, CH // 16, put_ridx2, 0)
    pltpu.sync_copy(rows0, degsh.at[ridx])

    def zdeg(i, _):
        degv[pl.ds(16 * i, 16)] = z
        return 0

    lax.fori_loop(0, NPAD // 16, zdeg, 0)
    plsc.subcore_barrier()

    @pl.when(c == 0)
    def _():
        _agg_pass(s, src1_hbm, dst1_hbm, x_hbm, acc,
                  srcb0, dstb0, rows0, sem0, srcb1, dstb1, rows1, sem1,
                  degv=degv)

    @pl.when(c == 1)
    def _():
        _agg_pass(s, src2_hbm, dst2_hbm, x_hbm, acc,
                  srcb0, dstb0, rows0, sem0, srcb1, dstb1, rows1, sem1,
                  degv=degv)

    plsc.subcore_barrier()

    def cpdeg(j, _):
        for g in range(D // 16):
            rows0[j, pl.ds(16 * g, 16)] = degv[pl.ds(128 * j + 16 * g, 16)]
        return 0

    lax.fori_loop(0, CH, cpdeg, 0)
    lax.fori_loop(0, CH // 16, put_ridx, 0)
    pltpu.sync_copy(rows0, degsh.at[ridx], add=True)

    def cpdeg2(j, _):
        for g in range(D // 16):
            rows0[j, pl.ds(16 * g, 16)] = degv[pl.ds(128 * (CH + j) + 16 * g,
                                                     16)]
        return 0

    lax.fori_loop(0, NRV - CH, cpdeg2, 0)

    def zrows(j, _):
        for g in range(D // 16):
            rows0[(NRV - CH) + j, pl.ds(16 * g, 16)] = z
        return 0

    lax.fori_loop(0, CH - (NRV - CH), zrows, 0)
    lax.fori_loop(0, CH // 16, put_ridx2, 0)
    pltpu.sync_copy(rows0, degsh.at[ridx], add=True)
    plsc.subcore_barrier()

    gidx[pl.ds(0, 16)] = jnp.minimum(lax.iota(i32, 16), NRV // NT - 1) + \
        (NRV // NT) * s
    pltpu.sync_copy(degsh.at[gidx], dgb)

    def flat(j, _):
        for g in range(D // 16):
            dfl[pl.ds(128 * j + 16 * g, 16)] = dgb[j, pl.ds(16 * g, 16)]
        return 0

    lax.fori_loop(0, NRV // NT, flat, 0)

    @pl.when(c == 0)
    def _():
        _finalize(s, acc, x_hbm, h1_hbm, None, rows0, ibuf, fidx, dfl=dfl,
                  inv_hbm=inv1_hbm)

    @pl.when(c == 1)
    def _():
        _finalize(s, acc, x_hbm, h2_hbm, None, rows0, ibuf, fidx, dfl=dfl,
                  inv_hbm=inv2_hbm)


def _layer2_body(h1_hbm, h2_hbm, src1_hbm, dst1_hbm, src2_hbm, dst2_hbm,
                 inv1_hbm, inv2_hbm, out_hbm,
                 srcb0, dstb0, rows0, srcb1, dstb1, rows1, ibuf, fidx,
                 acc, sem0, sem1):
    c = lax.axis_index("c")
    s = lax.axis_index("s")

    for p, table in enumerate((h1_hbm, h2_hbm)):
        _zero_acc(s, acc, rows0, srcb0, ch=CH2)
        plsc.subcore_barrier()

        @pl.when(c == 0)
        def _():
            _agg_pass(s, src1_hbm, dst1_hbm, table, acc,
                      srcb0, dstb0, rows0, sem0, srcb1, dstb1, rows1, sem1,
                      ch=CH2)

        @pl.when(c == 1)
        def _():
            _agg_pass(s, src2_hbm, dst2_hbm, table, acc,
                      srcb0, dstb0, rows0, sem0, srcb1, dstb1, rows1, sem1,
                      ch=CH2)

        plsc.subcore_barrier()

        @pl.when(c == 0)
        def _():
            _finalize(s, acc, table, out_hbm, D * p, rows0, ibuf, fidx,
                      inv_hbm=inv1_hbm)

        @pl.when(c == 1)
        def _():
            _finalize(s, acc, table, out_hbm, 2 * D + D * p, rows0, ibuf,
                      fidx, inv_hbm=inv2_hbm)

        plsc.subcore_barrier()


def _mesh():
    return plsc.VectorSubcoreMesh(core_axis_name="c", subcore_axis_name="s",
                                  num_cores=NC, num_subcores=NT)


_PARAMS = pltpu.CompilerParams(needs_layout_passes=False)


@jax.jit
def kernel(node_features, edge_index1, edge_index2):
    src1, dst1 = edge_index1[0], edge_index1[1]
    src2, dst2 = edge_index2[0], edge_index2[1]

    layer1 = pl.kernel(
        _layer1_body,
        out_type=[
            jax.ShapeDtypeStruct((N, D), f32),
            jax.ShapeDtypeStruct((N, D), f32),
            jax.ShapeDtypeStruct((NPAD, 16), f32),
            jax.ShapeDtypeStruct((NPAD, 16), f32),
        ],
        mesh=_mesh(),
        compiler_params=_PARAMS,
        scratch_types=[
            pltpu.VMEM((CH,), i32),
            pltpu.VMEM((CH,), i32),
            pltpu.VMEM((CH, D), f32),
            pltpu.VMEM((CH,), i32),
            pltpu.VMEM((CH,), i32),
            pltpu.VMEM((CH, D), f32),
            pltpu.VMEM((FCH, 16), f32),
            pltpu.VMEM((16,), i32),
            pltpu.VMEM((CH,), i32),
            pltpu.VMEM((16,), i32),
            pltpu.VMEM((NPAD,), f32),
            pltpu.VMEM((RPT,), f32),
            pltpu.VMEM((16, D), f32),
            pltpu.VMEM_SHARED((NPAD, D), f32),
            pltpu.VMEM_SHARED((128, D), f32),
            pltpu.SemaphoreType.DMA,
            pltpu.SemaphoreType.DMA,
        ],
        name="gin_layer1",
    )
    h1, h2, inv1, inv2 = layer1(node_features, src1, dst1, src2, dst2)

    layer2 = pl.kernel(
        _layer2_body,
        out_type=jax.ShapeDtypeStruct((N, 4 * D), f32),
        mesh=_mesh(),
        compiler_params=_PARAMS,
        scratch_types=[
            pltpu.VMEM((CH2,), i32),
            pltpu.VMEM((CH2,), i32),
            pltpu.VMEM((CH2, D), f32),
            pltpu.VMEM((CH2,), i32),
            pltpu.VMEM((CH2,), i32),
            pltpu.VMEM((CH2, D), f32),
            pltpu.VMEM((FCH, 16), f32),
            pltpu.VMEM((16,), i32),
            pltpu.VMEM_SHARED((NPAD, D), f32),
            pltpu.SemaphoreType.DMA,
            pltpu.SemaphoreType.DMA,
        ],
        name="gin_layer2",
    )
    return layer2(h1, h2, src1, dst1, src2, dst2, inv1, inv2)
